# baseline (speedup 1.0000x reference)
import jax
import jax.numpy as jnp
from jax.experimental import pallas as pl

_VOCAB = 100000
_VB = 2048
_NB = (_VOCAB + _VB - 1) // _VB
_H = 50


def _logits_kernel(hs_ref, emb_ref, tok_ref, fp_ref, pp_ref, t_ref, out_ref):
    j = pl.program_id(0)
    acc = jax.lax.dot_general(
        hs_ref[...], emb_ref[...],
        dimension_numbers=(((1,), (1,)), ((), ())),
        preferred_element_type=jnp.float32)
    cols = j * _VB + jax.lax.broadcasted_iota(jnp.int32, (1, _VB), 1)
    toks = tok_ref[...]
    counts = jnp.zeros(acc.shape, jnp.float32)
    for h in range(_H):
        counts = counts + (toks[:, h:h + 1] == cols).astype(jnp.float32)
    acc = acc - fp_ref[...] * counts
    acc = acc - pp_ref[...] * (counts > 0).astype(jnp.float32)
    t = jnp.maximum(t_ref[...], 1e-3)
    out_ref[...] = acc / t


def _blockmax_kernel(x_ref, o_ref):
    j = pl.program_id(0)
    cols = j * _VB + jax.lax.broadcasted_iota(jnp.int32, (1, _VB), 1)
    x = jnp.where(cols < _VOCAB, x_ref[...], -1e30)
    v = jnp.max(x, axis=1, keepdims=True)
    lane = jax.lax.broadcasted_iota(jnp.int32, (x.shape[0], _NB), 1)

    @pl.when(j == 0)
    def _():
        o_ref[...] = jnp.full((x.shape[0], _NB), -1e30, jnp.float32)

    o_ref[...] = jnp.where(lane == j, v, o_ref[...])


def _blocksum_kernel(x_ref, m_ref, o_ref):
    j = pl.program_id(0)
    cols = j * _VB + jax.lax.broadcasted_iota(jnp.int32, (1, _VB), 1)
    x = jnp.where(cols < _VOCAB, x_ref[...], -jnp.inf)
    v = jnp.sum(jnp.exp(x - m_ref[...]), axis=1, keepdims=True)
    lane = jax.lax.broadcasted_iota(jnp.int32, (x.shape[0], _NB), 1)

    @pl.when(j == 0)
    def _():
        o_ref[...] = jnp.zeros((x.shape[0], _NB), jnp.float32)

    o_ref[...] = jnp.where(lane == j, v, o_ref[...])


def _probs_kernel(x_ref, m_ref, ls_ref, p_ref, lp_ref):
    lp = x_ref[...] - m_ref[...] - ls_ref[...]
    p_ref[...] = jnp.exp(lp)
    lp_ref[...] = lp


def kernel(embedding, hidden_states, last_token_indices, output_tokens,
           presence_penalties, frequency_penalties, temperatures,
           top_ps, top_ks):
    b = last_token_indices.shape[0]
    hs = jnp.take(hidden_states, last_token_indices, axis=0)
    toks = output_tokens.astype(jnp.int32)
    fp = frequency_penalties[:, None]
    pp = presence_penalties[:, None]
    t = temperatures[:, None]

    logits = pl.pallas_call(
        _logits_kernel,
        grid=(_NB,),
        in_specs=[
            pl.BlockSpec((b, hs.shape[1]), lambda j: (0, 0)),
            pl.BlockSpec((_VB, embedding.shape[1]), lambda j: (j, 0)),
            pl.BlockSpec((b, _H), lambda j: (0, 0)),
            pl.BlockSpec((b, 1), lambda j: (0, 0)),
            pl.BlockSpec((b, 1), lambda j: (0, 0)),
            pl.BlockSpec((b, 1), lambda j: (0, 0)),
        ],
        out_specs=pl.BlockSpec((b, _VB), lambda j: (0, j)),
        out_shape=jax.ShapeDtypeStruct((b, _VOCAB), jnp.float32),
    )(hs, embedding, toks, fp, pp, t)

    sort_idx = jnp.argsort(-logits, axis=-1)
    logits_sort = jnp.take_along_axis(logits, sort_idx, axis=-1)
    probs_sort = jax.nn.softmax(logits_sort, axis=-1)
    probs_sum = jnp.cumsum(probs_sort, axis=-1)
    top_p_mask = (probs_sum - probs_sort) > top_ps[:, None]
    k = jnp.maximum(top_ks, 1)
    top_k_mask = jnp.arange(_VOCAB)[None, :] >= k[:, None]
    logits_sort = jnp.where(top_p_mask | top_k_mask, -1e9, logits_sort)
    inv_idx = jnp.argsort(sort_idx, axis=-1)
    masked = jnp.take_along_axis(logits_sort, inv_idx, axis=-1)

    bm = pl.pallas_call(
        _blockmax_kernel,
        grid=(_NB,),
        in_specs=[pl.BlockSpec((b, _VB), lambda j: (0, j))],
        out_specs=pl.BlockSpec((b, _NB), lambda j: (0, 0)),
        out_shape=jax.ShapeDtypeStruct((b, _NB), jnp.float32),
    )(masked)
    m = jnp.max(bm, axis=1, keepdims=True)

    bs = pl.pallas_call(
        _blocksum_kernel,
        grid=(_NB,),
        in_specs=[
            pl.BlockSpec((b, _VB), lambda j: (0, j)),
            pl.BlockSpec((b, 1), lambda j: (0, 0)),
        ],
        out_specs=pl.BlockSpec((b, _NB), lambda j: (0, 0)),
        out_shape=jax.ShapeDtypeStruct((b, _NB), jnp.float32),
    )(masked, m)
    ls = jnp.log(jnp.sum(bs, axis=1, keepdims=True))

    probs, logprobs = pl.pallas_call(
        _probs_kernel,
        grid=(_NB,),
        in_specs=[
            pl.BlockSpec((b, _VB), lambda j: (0, j)),
            pl.BlockSpec((b, 1), lambda j: (0, 0)),
            pl.BlockSpec((b, 1), lambda j: (0, 0)),
        ],
        out_specs=[
            pl.BlockSpec((b, _VB), lambda j: (0, j)),
            pl.BlockSpec((b, _VB), lambda j: (0, j)),
        ],
        out_shape=[
            jax.ShapeDtypeStruct((b, _VOCAB), jnp.float32),
            jax.ShapeDtypeStruct((b, _VOCAB), jnp.float32),
        ],
    )(masked, m, ls)

    next_tokens = jax.random.categorical(jax.random.key(42), masked, axis=-1)
    return next_tokens, probs, logprobs
